# baseline (speedup 1.0000x reference)
import functools

import jax
import jax.numpy as jnp
from jax import lax
from jax.experimental import pallas as pl
from jax.experimental.pallas import tpu as pltpu
from jax.experimental.pallas import tpu_sc as plsc

_NUM_CORES = 2
_NUM_SUBCORES = 16
_NW = _NUM_CORES * _NUM_SUBCORES
_CHUNK = 16
_NBUF = 4
_NPB = 2
_PF = _NBUF // 2
_LANES = 16


def _pe_add_body(x_hbm, len_hbm, pe_hbm, out_hbm, *scr,
                 rows_per_w, seq, d_emb, chunks):
  len_v = scr[0]
  xs = list(scr[1:1 + _NBUF])
  pb = list(scr[1 + _NBUF:1 + _NBUF + _NPB])
  idxv = list(scr[1 + _NBUF + _NPB:1 + _NBUF + 2 * _NPB])
  semx, semp, semo = scr[1 + _NBUF + 2 * _NPB:]

  wid = lax.axis_index("s") * _NUM_CORES + lax.axis_index("c")
  wpb = _NW // (rows_per_w * _NW // seq)
  b = wid // wpb
  c = wid % wpb
  groups = d_emb // _LANES

  def s_off(g):
    return (g * wpb + c) * _CHUNK

  pltpu.sync_copy(len_hbm.at[b], len_v)
  len_scalar = len_v[...][0]

  def x_copy(g, slot):
    return pltpu.make_async_copy(
        x_hbm.at[pl.ds(b * seq + s_off(g), _CHUNK)], xs[slot], semx.at[slot])

  half = _CHUNK // 2

  def out_half(g, slot, h):
    return pltpu.make_async_copy(
        xs[slot].at[pl.ds(h * half, half)],
        out_hbm.at[pl.ds(b * seq + s_off(g) + h * half, half)],
        semo.at[slot])

  def pe_start(g, slot):
    for j in range(_CHUNK // _LANES):
      idxv[slot][pl.ds(j * _LANES, _LANES)] = (
          s_off(g) + 1 + j * _LANES + lax.iota(jnp.int32, _LANES))
    pltpu.make_async_copy(
        pe_hbm.at[idxv[slot]], pb[slot], semp.at[slot]).start()

  def pe_wait(slot):
    pltpu.make_async_copy(
        pe_hbm.at[idxv[slot]], pb[slot], semp.at[slot]).wait()

  def add_needed(g):
    return s_off(g) + 1 <= len_scalar

  def add_rows(slot, pslot, lo, hi):
    def row_body(r, _):
      for j in range(groups):
        plsc.addupdate(xs[slot].at[r, pl.ds(j * _LANES, _LANES)],
                       pb[pslot][r, pl.ds(j * _LANES, _LANES)])
      return 0

    lax.fori_loop(lo, hi, row_body, 0)

  for k in range(_PF):
    x_copy(k, k).start()
  for k in range(_NPB):
    @pl.when(add_needed(k))
    def _(k=k):
      pe_start(k, k)

  def outer(i, _):
    for bb in range(_NBUF):
      g = i * _NBUF + bb
      nslot = (bb + _PF) % _NBUF
      pslot = bb % _NPB

      @pl.when(g >= _PF)
      def _():
        out_half(g - _PF, nslot, 0).wait()
        out_half(g - _PF, nslot, 1).wait()

      @pl.when(g + _PF < chunks)
      def _():
        x_copy(g + _PF, nslot).start()

      x_copy(g, bb).wait()

      nrows = jnp.minimum(len_scalar - s_off(g), _CHUNK)

      @pl.when(add_needed(g))
      def _():
        pe_wait(pslot)
        add_rows(bb, pslot, 0, jnp.minimum(nrows, half))

      out_half(g, bb, 0).start()

      @pl.when(add_needed(g))
      def _():
        add_rows(bb, pslot, half, jnp.maximum(nrows, half))

      out_half(g, bb, 1).start()

      @pl.when(jnp.logical_and(g + _NPB < chunks, add_needed(g + _NPB)))
      def _():
        pe_start(g + _NPB, pslot)
    return 0

  lax.fori_loop(0, chunks // _NBUF, outer, 0)
  for k in range(_PF):
    g = chunks - _PF + k
    out_half(g, g % _NBUF, 0).wait()
    out_half(g, g % _NBUF, 1).wait()


def kernel(x, lengths, pe_weight):
  n_batch, n_seq, d_emb = x.shape
  total_rows = n_batch * n_seq
  rows_per_w = total_rows // _NW
  chunks = rows_per_w // _CHUNK

  xf = x.reshape(total_rows, d_emb)
  lens16 = jnp.broadcast_to(
      lengths.astype(jnp.int32)[:, None], (n_batch, _LANES))

  mesh = plsc.VectorSubcoreMesh(core_axis_name="c", subcore_axis_name="s")
  body = functools.partial(
      _pe_add_body, rows_per_w=rows_per_w, seq=n_seq, d_emb=d_emb,
      chunks=chunks)
  out = pl.kernel(
      body,
      out_type=jax.ShapeDtypeStruct((total_rows, d_emb), jnp.float32),
      mesh=mesh,
      scratch_types=(
          [pltpu.VMEM((_LANES,), jnp.int32)]
          + [pltpu.VMEM((_CHUNK, d_emb), jnp.float32)] * _NBUF
          + [pltpu.VMEM((_CHUNK, d_emb), jnp.float32)] * _NPB
          + [pltpu.VMEM((_CHUNK,), jnp.int32)] * _NPB
          + [pltpu.SemaphoreType.DMA((_NBUF,)),
             pltpu.SemaphoreType.DMA((_NPB,)),
             pltpu.SemaphoreType.DMA((_NBUF,))]
      ),
  )(xf, lens16, pe_weight)
  return out.reshape(n_batch, n_seq, d_emb)


# speedup vs baseline: 1.3338x; 1.3338x over previous
import functools

import jax
import jax.numpy as jnp
from jax import lax
from jax.experimental import pallas as pl
from jax.experimental.pallas import tpu as pltpu
from jax.experimental.pallas import tpu_sc as plsc

_NUM_CORES = 2
_NUM_SUBCORES = 16
_NW = _NUM_CORES * _NUM_SUBCORES
_CHUNK = 16
_NBUF = 4
_NPB = 2
_PF = _NBUF // 2
_LANES = 16


def _pe_add_body(x_hbm, len_hbm, pe_hbm, out_hbm, *scr,
                 rows_per_w, seq, d_emb, chunks):
  len_v = scr[0]
  xs = list(scr[1:1 + _NBUF])
  pb = list(scr[1 + _NBUF:1 + _NBUF + _NPB])
  idxv = list(scr[1 + _NBUF + _NPB:1 + _NBUF + 2 * _NPB])
  semx, semp, semo = scr[1 + _NBUF + 2 * _NPB:]

  wid = lax.axis_index("s") * _NUM_CORES + lax.axis_index("c")
  wpb = _NW // (rows_per_w * _NW // seq)
  b = wid // wpb
  c = wid % wpb
  groups = d_emb // _LANES

  def s_off(g):
    return (g * wpb + c) * _CHUNK

  pltpu.sync_copy(len_hbm.at[b], len_v)
  len_scalar = len_v[...][0]

  def x_copy(g, slot):
    return pltpu.make_async_copy(
        x_hbm.at[pl.ds(b * seq + s_off(g), _CHUNK)], xs[slot], semx.at[slot])

  def out_copy(g, slot):
    return pltpu.make_async_copy(
        xs[slot], out_hbm.at[pl.ds(b * seq + s_off(g), _CHUNK)],
        semo.at[slot])

  def pe_start(g, slot):
    for j in range(_CHUNK // _LANES):
      idxv[slot][pl.ds(j * _LANES, _LANES)] = (
          s_off(g) + 1 + j * _LANES + lax.iota(jnp.int32, _LANES))
    pltpu.make_async_copy(
        pe_hbm.at[idxv[slot]], pb[slot], semp.at[slot]).start()

  def pe_wait(slot):
    pltpu.make_async_copy(
        pe_hbm.at[idxv[slot]], pb[slot], semp.at[slot]).wait()

  def add_needed(g):
    return s_off(g) + 1 <= len_scalar

  def add_rows(slot, pslot, lo, hi):
    def row_body(r, _):
      for j in range(groups):
        plsc.addupdate(xs[slot].at[r, pl.ds(j * _LANES, _LANES)],
                       pb[pslot][r, pl.ds(j * _LANES, _LANES)])
      return 0

    lax.fori_loop(lo, hi, row_body, 0)

  for k in range(_PF):
    x_copy(k, k).start()
  for k in range(_NPB):
    @pl.when(add_needed(k))
    def _(k=k):
      pe_start(k, k)

  def outer(i, _):
    for bb in range(_NBUF):
      g = i * _NBUF + bb
      nslot = (bb + _PF) % _NBUF
      pslot = bb % _NPB

      @pl.when(g >= _PF)
      def _():
        out_copy(g - _PF, nslot).wait()

      @pl.when(g + _PF < chunks)
      def _():
        x_copy(g + _PF, nslot).start()

      x_copy(g, bb).wait()

      @pl.when(add_needed(g))
      def _():
        pe_wait(pslot)
        nrows = jnp.minimum(len_scalar - s_off(g), _CHUNK)
        add_rows(bb, pslot, 0, nrows)

      out_copy(g, bb).start()

      @pl.when(jnp.logical_and(g + _NPB < chunks, add_needed(g + _NPB)))
      def _():
        pe_start(g + _NPB, pslot)
    return 0

  lax.fori_loop(0, chunks // _NBUF, outer, 0)
  for k in range(_PF):
    g = chunks - _PF + k
    out_copy(g, g % _NBUF).wait()


def kernel(x, lengths, pe_weight):
  n_batch, n_seq, d_emb = x.shape
  total_rows = n_batch * n_seq
  rows_per_w = total_rows // _NW
  chunks = rows_per_w // _CHUNK

  xf = x.reshape(total_rows, d_emb)
  lens16 = jnp.broadcast_to(
      lengths.astype(jnp.int32)[:, None], (n_batch, _LANES))

  mesh = plsc.VectorSubcoreMesh(core_axis_name="c", subcore_axis_name="s")
  body = functools.partial(
      _pe_add_body, rows_per_w=rows_per_w, seq=n_seq, d_emb=d_emb,
      chunks=chunks)
  out = pl.kernel(
      body,
      out_type=jax.ShapeDtypeStruct((total_rows, d_emb), jnp.float32),
      mesh=mesh,
      scratch_types=(
          [pltpu.VMEM((_LANES,), jnp.int32)]
          + [pltpu.VMEM((_CHUNK, d_emb), jnp.float32)] * _NBUF
          + [pltpu.VMEM((_CHUNK, d_emb), jnp.float32)] * _NPB
          + [pltpu.VMEM((_CHUNK,), jnp.int32)] * _NPB
          + [pltpu.SemaphoreType.DMA((_NBUF,)),
             pltpu.SemaphoreType.DMA((_NPB,)),
             pltpu.SemaphoreType.DMA((_NBUF,))]
      ),
  )(xf, lens16, pe_weight)
  return out.reshape(n_batch, n_seq, d_emb)
